# Initial kernel scaffold; baseline (speedup 1.0000x reference)
import jax
import jax.numpy as jnp
from jax.experimental import pallas as pl


def kernel(user_id, item_id, cate_id, hist_item_id, hist_cate_id, W_user_id, W_item_id, W_cate_id):
    raise NotImplementedError("write your pallas kernel here")


# speedup vs baseline: 2.2508x; 2.2508x over previous
import functools

import jax
import jax.numpy as jnp
from jax import lax
from jax.experimental import pallas as pl
from jax.experimental.pallas import tpu as pltpu
from jax.experimental.pallas import tpu_sc as plsc

B = 4096
L = 200
D = 32
NC = 2
NS = 16
NW = NC * NS

BH = B * L
ROWS_B = B // NW
HID_ROWS = BH // (NW * 128)
CHUNK_IR = 8
N_CHUNKS = HID_ROWS // CHUNK_IR


def _body(uid, iid, cid, hid, hcd, w_user, w_item, w_cate,
          user_out, item_out, hist_out,
          idx_b, rows_b, idx_h, rows_h, sem):
    wid = lax.axis_index("s") * NC + lax.axis_index("c")

    base = wid * ROWS_B

    def small_lookup(ids2d, table, out_ref, col):
        pltpu.sync_copy(ids2d.at[pl.ds(wid, 1)], idx_b)
        pltpu.async_copy(table.at[idx_b.at[0]], rows_b, sem).wait()
        pltpu.sync_copy(rows_b, out_ref.at[pl.ds(base, ROWS_B), pl.ds(col, D)])

    small_lookup(uid, w_user, user_out, 0)
    small_lookup(iid, w_item, item_out, 0)
    small_lookup(cid, w_cate, item_out, D)

    def hist_table(ids2d, table, col):
        def chunk(ci, carry):
            ir = wid * HID_ROWS + ci * CHUNK_IR
            pltpu.sync_copy(ids2d.at[pl.ds(ir, CHUNK_IR)], idx_h)
            cps = [
                pltpu.async_copy(
                    table.at[idx_h.at[j]],
                    rows_h.at[pl.ds(j * 128, 128)],
                    sem,
                )
                for j in range(CHUNK_IR)
            ]
            for cp in cps:
                cp.wait()
            pltpu.sync_copy(
                rows_h,
                hist_out.at[pl.ds(ir * 128, CHUNK_IR * 128), pl.ds(col, D)],
            )
            return carry

        lax.fori_loop(0, N_CHUNKS, chunk, 0)

    hist_table(hid, w_item, 0)
    hist_table(hcd, w_cate, D)


@functools.partial(jax.jit, static_argnums=())
def _run(uid, iid, cid, hid, hcd, w_user, w_item, w_cate):
    kern = pl.kernel(
        _body,
        out_type=[
            jax.ShapeDtypeStruct((B, D), jnp.float32),
            jax.ShapeDtypeStruct((B, 2 * D), jnp.float32),
            jax.ShapeDtypeStruct((BH, 2 * D), jnp.float32),
        ],
        mesh=plsc.VectorSubcoreMesh(core_axis_name="c", subcore_axis_name="s"),
        compiler_params=pltpu.CompilerParams(use_tc_tiling_on_sc=False),
        scratch_types=[
            pltpu.VMEM((1, 128), jnp.int32),
            pltpu.VMEM((128, D), jnp.float32),
            pltpu.VMEM((CHUNK_IR, 128), jnp.int32),
            pltpu.VMEM((CHUNK_IR * 128, D), jnp.float32),
            pltpu.SemaphoreType.DMA,
        ],
    )
    return kern(uid, iid, cid, hid, hcd, w_user, w_item, w_cate)


def kernel(user_id, item_id, cate_id, hist_item_id, hist_cate_id,
           W_user_id, W_item_id, W_cate_id):
    uid = user_id.astype(jnp.int32).reshape(NW, ROWS_B)
    iid = item_id.astype(jnp.int32).reshape(NW, ROWS_B)
    cid = cate_id.astype(jnp.int32).reshape(NW, ROWS_B)
    hid = hist_item_id.astype(jnp.int32).reshape(BH // 128, 128)
    hcd = hist_cate_id.astype(jnp.int32).reshape(BH // 128, 128)
    user_emb, item_emb, hist_flat = _run(
        uid, iid, cid, hid, hcd, W_user_id, W_item_id, W_cate_id)
    return user_emb, item_emb, hist_flat.reshape(B, L, 2 * D)


# speedup vs baseline: 2.3282x; 1.0344x over previous
import functools

import jax
import jax.numpy as jnp
from jax import lax
from jax.experimental import pallas as pl
from jax.experimental.pallas import tpu as pltpu
from jax.experimental.pallas import tpu_sc as plsc

B = 4096
L = 200
D = 32
NC = 2
NS = 16
NW = NC * NS

BH = B * L
ROWS_B = B // NW
HID_ROWS = BH // (NW * 128)
K = 5
CR = K * 128
NCHUNK = HID_ROWS // K
NBODY = NCHUNK // 2


def _body(uid, iid, cid, hid, hcd, w_user, w_item, w_cate,
          user_out, item_out, hist_out,
          idx_b, rows_b, idx_i, idx_c, rows_i, rows_c,
          sem_b, sem_ii0, sem_ii1, sem_ic0, sem_ic1,
          sem_gi0, sem_gi1, sem_gc0, sem_gc1,
          sem_wi0, sem_wi1, sem_wc0, sem_wc1):
    wid = lax.axis_index("s") * NC + lax.axis_index("c")
    hrow0 = wid * HID_ROWS

    base = wid * ROWS_B

    def small_lookup(ids2d, table, out_ref, col):
        pltpu.sync_copy(ids2d.at[pl.ds(wid, 1)], idx_b)
        pltpu.async_copy(table.at[idx_b.at[0]], rows_b, sem_b).wait()
        pltpu.sync_copy(rows_b, out_ref.at[pl.ds(base, ROWS_B), pl.ds(col, D)])

    small_lookup(uid, w_user, user_out, 0)
    small_lookup(iid, w_item, item_out, 0)
    small_lookup(cid, w_cate, item_out, D)

    def idx_cp(ids2d, idx_ref, slot, c, sem):
        return pltpu.make_async_copy(
            ids2d.at[pl.ds(hrow0 + c * K, K)],
            idx_ref.at[pl.ds(slot * K, K)], sem)

    def fires(table, idx_ref, slot, rows_ref, sem):
        return [
            pltpu.async_copy(
                table.at[idx_ref.at[slot * K + j]],
                rows_ref.at[pl.ds((slot * K + j) * 128, 128)], sem)
            for j in range(K)
        ]

    def write_cp(rows_ref, slot, c, col, sem):
        return pltpu.make_async_copy(
            rows_ref.at[pl.ds(slot * CR, CR)],
            hist_out.at[pl.ds((hrow0 + c * K) * 128, CR), pl.ds(col, D)],
            sem)

    idx_cp(hid, idx_i, 0, 0, sem_ii0).start()
    idx_cp(hcd, idx_c, 0, 0, sem_ic0).start()
    idx_cp(hid, idx_i, 1, 1, sem_ii1).start()
    idx_cp(hcd, idx_c, 1, 1, sem_ic1).start()

    def body(g, carry):
        c0 = 2 * g
        c1 = 2 * g + 1
        idx_cp(hid, idx_i, 0, c0, sem_ii0).wait()

        @pl.when(g > 0)
        def _():
            write_cp(rows_i, 0, c0 - 2, 0, sem_wi0).wait()
        gi0 = fires(w_item, idx_i, 0, rows_i, sem_gi0)

        idx_cp(hcd, idx_c, 0, c0, sem_ic0).wait()

        @pl.when(g > 0)
        def _():
            write_cp(rows_c, 0, c0 - 2, D, sem_wc0).wait()
        gc0 = fires(w_cate, idx_c, 0, rows_c, sem_gc0)

        idx_cp(hid, idx_i, 1, c1, sem_ii1).wait()

        @pl.when(g > 0)
        def _():
            write_cp(rows_i, 1, c1 - 2, 0, sem_wi1).wait()
        gi1 = fires(w_item, idx_i, 1, rows_i, sem_gi1)

        idx_cp(hcd, idx_c, 1, c1, sem_ic1).wait()

        @pl.when(g > 0)
        def _():
            write_cp(rows_c, 1, c1 - 2, D, sem_wc1).wait()
        gc1 = fires(w_cate, idx_c, 1, rows_c, sem_gc1)

        for cp in gi0:
            cp.wait()
        write_cp(rows_i, 0, c0, 0, sem_wi0).start()

        @pl.when(g < NBODY - 1)
        def _():
            idx_cp(hid, idx_i, 0, c0 + 2, sem_ii0).start()

        for cp in gc0:
            cp.wait()
        write_cp(rows_c, 0, c0, D, sem_wc0).start()

        @pl.when(g < NBODY - 1)
        def _():
            idx_cp(hcd, idx_c, 0, c0 + 2, sem_ic0).start()

        for cp in gi1:
            cp.wait()
        write_cp(rows_i, 1, c1, 0, sem_wi1).start()

        @pl.when(g < NBODY - 1)
        def _():
            idx_cp(hid, idx_i, 1, c1 + 2, sem_ii1).start()

        for cp in gc1:
            cp.wait()
        write_cp(rows_c, 1, c1, D, sem_wc1).start()

        @pl.when(g < NBODY - 1)
        def _():
            idx_cp(hcd, idx_c, 1, c1 + 2, sem_ic1).start()

        return carry

    lax.fori_loop(0, NBODY, body, 0)

    write_cp(rows_i, 0, NCHUNK - 2, 0, sem_wi0).wait()
    write_cp(rows_c, 0, NCHUNK - 2, D, sem_wc0).wait()
    write_cp(rows_i, 1, NCHUNK - 1, 0, sem_wi1).wait()
    write_cp(rows_c, 1, NCHUNK - 1, D, sem_wc1).wait()


@jax.jit
def _run(uid, iid, cid, hid, hcd, w_user, w_item, w_cate):
    kern = pl.kernel(
        _body,
        out_type=[
            jax.ShapeDtypeStruct((B, D), jnp.float32),
            jax.ShapeDtypeStruct((B, 2 * D), jnp.float32),
            jax.ShapeDtypeStruct((BH, 2 * D), jnp.float32),
        ],
        mesh=plsc.VectorSubcoreMesh(core_axis_name="c", subcore_axis_name="s"),
        compiler_params=pltpu.CompilerParams(use_tc_tiling_on_sc=False),
        scratch_types=[
            pltpu.VMEM((1, 128), jnp.int32),
            pltpu.VMEM((128, D), jnp.float32),
            pltpu.VMEM((2 * K, 128), jnp.int32),
            pltpu.VMEM((2 * K, 128), jnp.int32),
            pltpu.VMEM((2 * CR, D), jnp.float32),
            pltpu.VMEM((2 * CR, D), jnp.float32),
        ] + [pltpu.SemaphoreType.DMA] * 13,
    )
    return kern(uid, iid, cid, hid, hcd, w_user, w_item, w_cate)


def kernel(user_id, item_id, cate_id, hist_item_id, hist_cate_id,
           W_user_id, W_item_id, W_cate_id):
    uid = user_id.astype(jnp.int32).reshape(NW, ROWS_B)
    iid = item_id.astype(jnp.int32).reshape(NW, ROWS_B)
    cid = cate_id.astype(jnp.int32).reshape(NW, ROWS_B)
    hid = hist_item_id.astype(jnp.int32).reshape(BH // 128, 128)
    hcd = hist_cate_id.astype(jnp.int32).reshape(BH // 128, 128)
    user_emb, item_emb, hist_flat = _run(
        uid, iid, cid, hid, hcd, W_user_id, W_item_id, W_cate_id)
    return user_emb, item_emb, hist_flat.reshape(B, L, 2 * D)
